# baseline (speedup 1.0000x reference)
import functools

import jax
import jax.numpy as jnp
from jax import lax
from jax.experimental import pallas as pl
from jax.experimental.pallas import tpu as pltpu
from jax.experimental.pallas import tpu_sc as plsc

N = 50000
E = 800000
HID = 64
HEADS = 4
D = HEADS * HID

NC, NS, L = 2, 16, 16
NHALF = N // 2
NQ = N // 4
NP = 25088
RPT = NP // NS
NP4 = 12544
RPT4 = NP4 // NS
BLKA = 64
BLKB = 32
SB = 2048
NBA = SB // BLKA
NBB = SB // BLKB
CAP = 53248
STRIPE = E // NS
CH = 2000
ZR = 56
DZR = 392

_mesh = lambda: plsc.VectorSubcoreMesh(
    core_axis_name="c", subcore_axis_name="s", num_cores=NC, num_subcores=NS)


ROWS = 1000
_P = jax.lax.Precision.HIGHEST


def _elu(x):
    return jnp.where(x > 0, x, jnp.exp(x) - 1.0)


def _mm(a, w, b):
    return jnp.dot(a, w, precision=_P, preferred_element_type=jnp.float32) + b


def _enc_body(x_ref, w1, b1, w2, b2, o_ref):
    h = _elu(_mm(x_ref[...], w1[...], b1[...]))
    o_ref[...] = _elu(_mm(h, w2[...], b2[...]))


def _qkv_body(h_ref, wq, bq, wk, bk, wv, bv, ws, bs, q_ref, k_ref, v_ref, s_ref):
    h = h_ref[...]
    q_ref[...] = _mm(h, wq[...], bq[...])
    k_ref[...] = _mm(h, wk[...], bk[...])
    v_ref[...] = _mm(h, wv[...], bv[...])
    s_ref[...] = _mm(h, ws[...], bs[...])


def _comb_qkv_body(a_ref, hs_ref, wq, bq, wk, bk, wv, bv, ws, bs,
                   q_ref, k_ref, v_ref, s_ref):
    h = _elu(a_ref[...] + hs_ref[...])
    q_ref[...] = _mm(h, wq[...], bq[...])
    k_ref[...] = _mm(h, wk[...], bk[...])
    v_ref[...] = _mm(h, wv[...], bv[...])
    s_ref[...] = _mm(h, ws[...], bs[...])


def _out_body(a_ref, hs_ref, w1, b1, w2, b2, w3, b3, o_ref):
    h = _elu(a_ref[...] + hs_ref[...])
    o = _elu(_mm(h, w1[...], b1[...]))
    o = _elu(_mm(o, w2[...], b2[...]))
    o_ref[...] = _mm(o, w3[...], b3[...])


def _row_spec(cols):
    return pl.BlockSpec((ROWS, cols), lambda i: (i, 0))


def _full_spec(r, c):
    return pl.BlockSpec((r, c), lambda i: (0, 0))


def _wspecs(shapes):
    return [_full_spec(*s) for s in shapes]


def _tc_enc(x, w1, b1, w2, b2):
    return pl.pallas_call(
        _enc_body,
        grid=(N // ROWS,),
        in_specs=[_row_spec(8)] + _wspecs([(8, HID), (1, HID), (HID, HID), (1, HID)]),
        out_specs=_row_spec(HID),
        out_shape=jax.ShapeDtypeStruct((N, HID), jnp.float32),
    )(x, w1, b1.reshape(1, -1), w2, b2.reshape(1, -1))


def _qkv_shapes():
    return [jax.ShapeDtypeStruct((N, D), jnp.float32)] * 3 + [
        jax.ShapeDtypeStruct((N, HID), jnp.float32)]


def _lp_args(lp):
    return (lp['Wq'], lp['bq'].reshape(1, -1), lp['Wk'], lp['bk'].reshape(1, -1),
            lp['Wv'], lp['bv'].reshape(1, -1), lp['Wskip'], lp['bskip'].reshape(1, -1))


_LPW = [(HID, D), (1, D), (HID, D), (1, D), (HID, D), (1, D), (HID, HID), (1, HID)]


def _tc_qkv(h, lp):
    return pl.pallas_call(
        _qkv_body,
        grid=(N // ROWS,),
        in_specs=[_row_spec(HID)] + _wspecs(_LPW),
        out_specs=[_row_spec(D)] * 3 + [_row_spec(HID)],
        out_shape=_qkv_shapes(),
    )(h, *_lp_args(lp))


def _tc_comb_qkv(agg, hs, lp):
    return pl.pallas_call(
        _comb_qkv_body,
        grid=(N // ROWS,),
        in_specs=[_row_spec(HID), _row_spec(HID)] + _wspecs(_LPW),
        out_specs=[_row_spec(D)] * 3 + [_row_spec(HID)],
        out_shape=_qkv_shapes(),
    )(agg, hs, *_lp_args(lp))


def _tc_out(agg, hs, w1, b1, w2, b2, w3, b3):
    return pl.pallas_call(
        _out_body,
        grid=(N // ROWS,),
        in_specs=[_row_spec(HID), _row_spec(HID)] + _wspecs(
            [(HID, 64), (1, 64), (64, 32), (1, 32), (32, 8), (1, 8)]),
        out_specs=_row_spec(8),
        out_shape=jax.ShapeDtypeStruct((N, 8), jnp.float32),
    )(agg, hs, w1, b1.reshape(1, -1), w2, b2.reshape(1, -1), w3, b3.reshape(1, -1))


def _filter_body(src_hbm, dst_hbm, ksrc_hbm, kdst_hbm, cnt_hbm,
                 srcst, dstst, ksrc_st, kdst_st, cst):
    c = lax.axis_index("c")
    s = lax.axis_index("s")
    cbase = c * NHALF
    lane = lax.broadcasted_iota(jnp.int32, (L,), 0)
    stripe0 = s * STRIPE
    zi = jnp.zeros((L,), jnp.int32)

    for q in range(2):
        qbase = cbase + q * NQ

        def chunk(ci, cur):
            pltpu.sync_copy(src_hbm.at[pl.ds(stripe0 + ci * CH, CH)], srcst)
            pltpu.sync_copy(dst_hbm.at[pl.ds(stripe0 + ci * CH, CH)], dstst)

            def grp(gi, cur):
                s16 = srcst[pl.ds(gi * L, L)]
                d16 = dstst[pl.ds(gi * L, L)]
                dl = d16 - qbase
                m = (dl >= 0) & (dl < NQ)
                mi = jnp.where(m, 1, 0)
                pos = cur + plsc.cumsum(mi) - mi
                plsc.store_scatter(ksrc_st, [pos], s16, mask=m)
                plsc.store_scatter(kdst_st, [pos], d16, mask=m)
                return cur + plsc.all_reduce_population_count(m)

            return lax.fori_loop(0, CH // L, grp, cur)

        cur = lax.fori_loop(0, STRIPE // CH, chunk, jnp.zeros((L,), jnp.int32))
        cnt = jnp.max(cur)
        cntp = ((cnt + SB - 1) // SB) * SB
        trash = cbase + NHALF

        def dumm(i, _):
            pos = cnt + i * L + lane
            m = pos < cntp
            plsc.store_scatter(ksrc_st, [pos], zi, mask=m)
            plsc.store_scatter(kdst_st, [pos], lane * 0 + trash, mask=m)
            return 0
        lax.fori_loop(0, SB // L, dumm, 0)

        rbase = ((c * NS + s) * 2 + q) * CAP
        pltpu.sync_copy(ksrc_st, ksrc_hbm.at[pl.ds(rbase, CAP)])
        pltpu.sync_copy(kdst_st, kdst_hbm.at[pl.ds(rbase, CAP)])
        cst[...] = lane * 0 + cntp
        pltpu.sync_copy(cst, cnt_hbm.at[pl.ds(((c * NS + s) * 2 + q) * L, L)])


def _sc_filter(src, dst):
    f = pl.kernel(
        _filter_body,
        out_type=[jax.ShapeDtypeStruct((NC * NS * 2 * CAP,), jnp.int32),
                  jax.ShapeDtypeStruct((NC * NS * 2 * CAP,), jnp.int32),
                  jax.ShapeDtypeStruct((NC * NS * 2 * L,), jnp.int32)],
        mesh=_mesh(),
        compiler_params=pltpu.CompilerParams(
            needs_layout_passes=False, use_tc_tiling_on_sc=False),
        scratch_types=[pltpu.VMEM((CH,), jnp.int32), pltpu.VMEM((CH,), jnp.int32),
                       pltpu.VMEM((CAP,), jnp.int32), pltpu.VMEM((CAP,), jnp.int32),
                       pltpu.VMEM((L,), jnp.int32)],
    )
    return f(src, dst)


def _alpha_body(q_hbm, k_hbm, ksrc_hbm, kdst_hbm, cnt_hbm,
                ex_hbm, rez_hbm,
                srcsb, dstsb, qix, six, dl, exsb, exr0, exr1,
                bufa0, bufb0, bufa1, bufb1, dzc, prix, cntb, denom_sp,
                smq0, smk0, smq1, smk1, ssc0, ssc1, smx):
    c = lax.axis_index("c")
    s = lax.axis_index("s")
    cbase = c * NHALF
    lane = lax.broadcasted_iota(jnp.int32, (L,), 0)
    r0 = s * RPT
    zf = jnp.zeros((L,), jnp.float32)
    zi = jnp.zeros((L,), jnp.int32)

    def z1(i, _):
        dzc[i, :] = zf
        return 0
    lax.fori_loop(0, DZR, z1, 0)
    for j in range(RPT // DZR):
        pltpu.sync_copy(dzc, denom_sp.at[pl.ds(r0 + j * DZR, DZR)])

    def z2(i, _):
        exr0[i, :] = zf
        exr1[i, :] = zf
        return 0
    lax.fori_loop(0, BLKA, z2, 0)

    def z3(i, _):
        prix[pl.ds(i * L, L)] = zi + (NP - 1)
        return 0
    lax.fori_loop(0, BLKA // L, z3, 0)
    plsc.subcore_barrier()

    def gissue(b, ba, bb, sq, sk):
        off = b * BLKA
        pltpu.async_copy(q_hbm.at[qix.at[pl.ds(off, BLKA)]], ba, sq)
        pltpu.async_copy(k_hbm.at[six.at[pl.ds(off, BLKA)]], bb, sk)

    def gwait(ba, bb, sq, sk):
        pltpu.make_async_copy(q_hbm.at[pl.ds(0, BLKA)], ba, sq).wait()
        pltpu.make_async_copy(k_hbm.at[pl.ds(0, BLKA)], bb, sk).wait()

    def sscdrain(exr, ssc):
        pltpu.make_async_copy(rez_hbm.at[pl.ds(0, BLKA)], exr, ssc).wait()

    def compute(b, ba, bb, exr, ssc):
        sscdrain(exr, ssc)
        boff = b * BLKA

        def gh(t, _):
            g = t // HEADS
            h = t % HEADS
            row = g * L + lane
            col0 = h * HID

            def cc(c2, acc):
                for u in range(8):
                    col = col0 + c2 * 8 + u
                    acc = acc + (plsc.load_gather(ba, [row, lane * 0 + col]) *
                                 plsc.load_gather(bb, [row, lane * 0 + col]))
                return acc

            acc = lax.fori_loop(0, HID // 8, cc, zf)
            ex = jnp.exp(acc * 0.125)
            exsb[h, pl.ds(boff + g * L, L)] = ex
            plsc.store_scatter(exr, [row, lane * 0 + h], ex)
            return 0

        lax.fori_loop(0, (BLKA // L) * HEADS, gh, 0)
        pltpu.async_copy(exr, denom_sp.at[dl.at[b]], ssc, add=True)

    for q in range(2):
        rbase = ((c * NS + s) * 2 + q) * CAP
        pltpu.sync_copy(cnt_hbm.at[pl.ds(((c * NS + s) * 2 + q) * L, L)], cntb)
        cnt = jnp.max(cntb[...])
        nsb = cnt // SB

        def do_sb(k, _):
            sbase = k * SB
            cm1 = pltpu.async_copy(
                ksrc_hbm.at[pl.ds(rbase + sbase, SB)], srcsb, smx)
            cm2 = pltpu.async_copy(
                kdst_hbm.at[pl.ds(rbase + sbase, SB)], dstsb, smx)
            pltpu.async_copy(exr0, denom_sp.at[prix], ssc0, add=True)
            pltpu.async_copy(exr1, denom_sp.at[prix], ssc1, add=True)
            cm1.wait()
            cm2.wait()

            def idx(i, _):
                d16 = dstsb[pl.ds(i * L, L)]
                s16 = srcsb[pl.ds(i * L, L)]
                qix[pl.ds(i * L, L)] = jnp.minimum(jnp.maximum(d16, 0), N - 1)
                six[pl.ds(i * L, L)] = jnp.minimum(jnp.maximum(s16, 0), N - 1)
                dl[i // (BLKA // L), pl.ds((i % (BLKA // L)) * L, L)] = d16 - cbase
                return 0
            lax.fori_loop(0, SB // L, idx, 0)

            gissue(0, bufa0, bufb0, smq0, smk0)

            def pair(p, _):
                b0 = 2 * p
                gissue(b0 + 1, bufa1, bufb1, smq1, smk1)
                gwait(bufa0, bufb0, smq0, smk0)
                compute(b0, bufa0, bufb0, exr0, ssc0)
                gissue(jnp.minimum(b0 + 2, NBA - 1), bufa0, bufb0, smq0, smk0)
                gwait(bufa1, bufb1, smq1, smk1)
                compute(b0 + 1, bufa1, bufb1, exr1, ssc1)
                return 0
            lax.fori_loop(0, NBA // 2, pair, 0)
            gwait(bufa0, bufb0, smq0, smk0)
            sscdrain(exr0, ssc0)
            sscdrain(exr1, ssc1)

            for h in range(HEADS):
                pltpu.async_copy(
                    exsb.at[h],
                    ex_hbm.at[pl.ds((((h * NC + c) * NS + s) * 2 + q) * CAP
                                    + sbase, SB)], smx)
            for h in range(HEADS):
                pltpu.make_async_copy(
                    ex_hbm.at[pl.ds(h * SB, SB)], exsb.at[h], smx).wait()
            return 0
        lax.fori_loop(0, nsb, do_sb, 0)

    plsc.subcore_barrier()

    for j in range(RPT // DZR):
        pltpu.sync_copy(denom_sp.at[pl.ds(r0 + j * DZR, DZR)], dzc)

        def rz(i, _):
            dzc[i, :] = 0.25 / (dzc[i, :] + 1e-16)
            return 0
        lax.fori_loop(0, DZR, rz, 0)
        pltpu.sync_copy(dzc, rez_hbm.at[pl.ds(c * NP + r0 + j * DZR, DZR)])


def _sc_alpha(q, k, ksrc, kdst, cnts):
    f = pl.kernel(
        _alpha_body,
        out_type=[jax.ShapeDtypeStruct((HEADS * NC * NS * 2 * CAP,), jnp.float32),
                  jax.ShapeDtypeStruct((NC * NP, L), jnp.float32)],
        mesh=_mesh(),
        compiler_params=pltpu.CompilerParams(
            needs_layout_passes=False, use_tc_tiling_on_sc=False),
        scratch_types=[
            pltpu.VMEM((SB,), jnp.int32),
            pltpu.VMEM((SB,), jnp.int32),
            pltpu.VMEM((SB,), jnp.int32),
            pltpu.VMEM((SB,), jnp.int32),
            pltpu.VMEM((NBA, BLKA), jnp.int32),
            pltpu.VMEM((HEADS, SB), jnp.float32),
            pltpu.VMEM((BLKA, L), jnp.float32),
            pltpu.VMEM((BLKA, L), jnp.float32),
            pltpu.VMEM((BLKA, D), jnp.float32),
            pltpu.VMEM((BLKA, D), jnp.float32),
            pltpu.VMEM((BLKA, D), jnp.float32),
            pltpu.VMEM((BLKA, D), jnp.float32),
            pltpu.VMEM((DZR, L), jnp.float32),
            pltpu.VMEM((BLKA,), jnp.int32),
            pltpu.VMEM((L,), jnp.int32),
            pltpu.VMEM_SHARED((NP, L), jnp.float32),
            pltpu.SemaphoreType.DMA, pltpu.SemaphoreType.DMA,
            pltpu.SemaphoreType.DMA, pltpu.SemaphoreType.DMA,
            pltpu.SemaphoreType.DMA, pltpu.SemaphoreType.DMA,
            pltpu.SemaphoreType.DMA,
        ],
    )
    return f(q, k, ksrc, kdst, cnts)


def _agg_body(v_hbm, ksrc_hbm, kdst_hbm, cnt_hbm, ex_hbm, rez_hbm,
              out_hbm,
              srcsb, dstsb, six, rix, dl, exsb, wv0, wv1,
              bufv0, bufv1, rez0, rez1, za, prix, cntb, agg_sp,
              smv0, smr0, smv1, smr1, ssc0, ssc1, smx):
    c = lax.axis_index("c")
    s = lax.axis_index("s")
    cbase = c * NHALF
    lane = lax.broadcasted_iota(jnp.int32, (L,), 0)
    r0 = s * RPT4
    zf = jnp.zeros((L,), jnp.float32)
    zi = jnp.zeros((L,), jnp.int32)

    def z1(i, _):
        for j in range(HID // L):
            za[i, pl.ds(j * L, L)] = zf
        return 0
    lax.fori_loop(0, ZR, z1, 0)

    def z2(i, _):
        for j in range(HID // L):
            wv0[i, pl.ds(j * L, L)] = zf
            wv1[i, pl.ds(j * L, L)] = zf
        return 0
    lax.fori_loop(0, BLKB, z2, 0)

    def z3(i, _):
        prix[pl.ds(i * L, L)] = zi + (NP4 - 1)
        return 0
    lax.fori_loop(0, BLKB // L, z3, 0)

    def vissue(b, bv, rz_, sv, sr):
        off = b * BLKB
        pltpu.async_copy(v_hbm.at[six.at[pl.ds(off, BLKB)]], bv, sv)
        pltpu.async_copy(rez_hbm.at[rix.at[pl.ds(off, BLKB)]], rz_, sr)

    def vwait(bv, rz_, sv, sr):
        pltpu.make_async_copy(v_hbm.at[pl.ds(0, BLKB)], bv, sv).wait()
        pltpu.make_async_copy(rez_hbm.at[pl.ds(0, BLKB)], rz_, sr).wait()

    def sscdrain(wv, ssc):
        pltpu.make_async_copy(out_hbm.at[0, pl.ds(0, BLKB)], wv, ssc).wait()

    def compute(b, bv, rz_, wv, ssc):
        sscdrain(wv, ssc)
        boff = b * BLKB

        def g2(g, _):
            row = g * L + lane
            at = []
            for h in range(HEADS):
                e = exsb[h, pl.ds(boff + g * L, L)]
                at.append(e * plsc.load_gather(rz_, [row, lane * 0 + h]))

            def cc(c2, _2):
                for u in range(4):
                    col = c2 * 4 + u
                    acc = at[0] * plsc.load_gather(bv, [row, lane * 0 + col])
                    for h in range(1, HEADS):
                        acc = acc + at[h] * plsc.load_gather(
                            bv, [row, lane * 0 + (h * HID + col)])
                    plsc.store_scatter(wv, [row, lane * 0 + col], acc)
                return 0

            lax.fori_loop(0, HID // 4, cc, 0)
            return 0

        lax.fori_loop(0, BLKB // L, g2, 0)
        pltpu.async_copy(wv, agg_sp.at[dl.at[b]], ssc, add=True)

    for q in range(2):
        for j in range(RPT4 // ZR):
            pltpu.sync_copy(za, agg_sp.at[pl.ds(r0 + j * ZR, ZR)])
        plsc.subcore_barrier()

        qoff = q * NQ
        rbase = ((c * NS + s) * 2 + q) * CAP
        pltpu.sync_copy(cnt_hbm.at[pl.ds(((c * NS + s) * 2 + q) * L, L)], cntb)
        cnt = jnp.max(cntb[...])
        nsb = cnt // SB

        def do_sb(k, _):
            sbase = k * SB
            cm1 = pltpu.async_copy(
                ksrc_hbm.at[pl.ds(rbase + sbase, SB)], srcsb, smx)
            cm2 = pltpu.async_copy(
                kdst_hbm.at[pl.ds(rbase + sbase, SB)], dstsb, smx)
            for h in range(HEADS):
                pltpu.async_copy(
                    ex_hbm.at[pl.ds((((h * NC + c) * NS + s) * 2 + q) * CAP
                                    + sbase, SB)], exsb.at[h], smx)
            pltpu.async_copy(wv0, agg_sp.at[prix], ssc0, add=True)
            pltpu.async_copy(wv1, agg_sp.at[prix], ssc1, add=True)
            cm1.wait()
            cm2.wait()
            for h in range(HEADS):
                pltpu.make_async_copy(
                    ex_hbm.at[pl.ds(h * SB, SB)], exsb.at[h], smx).wait()

            def idx(i, _):
                d16 = dstsb[pl.ds(i * L, L)]
                s16 = srcsb[pl.ds(i * L, L)]
                six[pl.ds(i * L, L)] = jnp.minimum(jnp.maximum(s16, 0), N - 1)
                rix[pl.ds(i * L, L)] = d16 - cbase + c * NP
                dl[i // (BLKB // L), pl.ds((i % (BLKB // L)) * L, L)] = (
                    jnp.minimum(d16 - cbase - qoff, NP4 - 1))
                return 0
            lax.fori_loop(0, SB // L, idx, 0)

            vissue(0, bufv0, rez0, smv0, smr0)

            def pair(p, _):
                b0 = 2 * p
                vissue(b0 + 1, bufv1, rez1, smv1, smr1)
                vwait(bufv0, rez0, smv0, smr0)
                compute(b0, bufv0, rez0, wv0, ssc0)
                vissue(jnp.minimum(b0 + 2, NBB - 1), bufv0, rez0, smv0, smr0)
                vwait(bufv1, rez1, smv1, smr1)
                compute(b0 + 1, bufv1, rez1, wv1, ssc1)
                return 0
            lax.fori_loop(0, NBB // 2, pair, 0)
            vwait(bufv0, rez0, smv0, smr0)
            sscdrain(wv0, ssc0)
            sscdrain(wv1, ssc1)
            return 0
        lax.fori_loop(0, nsb, do_sb, 0)
        plsc.subcore_barrier()

        for j in range(RPT4 // ZR):
            pltpu.sync_copy(agg_sp.at[pl.ds(r0 + j * ZR, ZR)], za)
            pltpu.sync_copy(za, out_hbm.at[c * 2 + q, pl.ds(r0 + j * ZR, ZR)])

        def z1b(i, _):
            for j in range(HID // L):
                za[i, pl.ds(j * L, L)] = zf
            return 0
        lax.fori_loop(0, ZR, z1b, 0)
        plsc.subcore_barrier()


def _sc_agg(v, ksrc, kdst, cnts, ex, rez):
    f = pl.kernel(
        _agg_body,
        out_type=[jax.ShapeDtypeStruct((NC * 2, NP4, HID), jnp.float32)],
        mesh=_mesh(),
        compiler_params=pltpu.CompilerParams(
            needs_layout_passes=False, use_tc_tiling_on_sc=False),
        scratch_types=[
            pltpu.VMEM((SB,), jnp.int32),
            pltpu.VMEM((SB,), jnp.int32),
            pltpu.VMEM((SB,), jnp.int32),
            pltpu.VMEM((SB,), jnp.int32),
            pltpu.VMEM((NBB, BLKB), jnp.int32),
            pltpu.VMEM((HEADS, SB), jnp.float32),
            pltpu.VMEM((BLKB, HID), jnp.float32),
            pltpu.VMEM((BLKB, HID), jnp.float32),
            pltpu.VMEM((BLKB, D), jnp.float32),
            pltpu.VMEM((BLKB, D), jnp.float32),
            pltpu.VMEM((BLKB, L), jnp.float32),
            pltpu.VMEM((BLKB, L), jnp.float32),
            pltpu.VMEM((ZR, HID), jnp.float32),
            pltpu.VMEM((BLKB,), jnp.int32),
            pltpu.VMEM((L,), jnp.int32),
            pltpu.VMEM_SHARED((NP4, HID), jnp.float32),
            pltpu.SemaphoreType.DMA, pltpu.SemaphoreType.DMA,
            pltpu.SemaphoreType.DMA, pltpu.SemaphoreType.DMA,
            pltpu.SemaphoreType.DMA, pltpu.SemaphoreType.DMA,
            pltpu.SemaphoreType.DMA,
        ],
    )
    (out,) = f(v, ksrc, kdst, cnts, ex, rez)
    return jnp.concatenate([out[0, :NQ], out[1, :NQ],
                            out[2, :NQ], out[3, :NQ]], axis=0)


def _sc_layer(q, k, v, ksrc, kdst, cnts):
    ex, rez = _sc_alpha(q, k, ksrc, kdst, cnts)
    return _sc_agg(v, ksrc, kdst, cnts, ex, rez)


def kernel(x, edge_index, params):
    src = edge_index[0]
    dst = edge_index[1]
    ksrc, kdst, cnts = _sc_filter(src, dst)
    h = _tc_enc(x, params['enc_W1'], params['enc_b1'],
                params['enc_W2'], params['enc_b2'])
    layers = params['layers']
    q, k, v, hs = _tc_qkv(h, layers[0])
    agg = _sc_layer(q, k, v, ksrc, kdst, cnts)
    for lp in layers[1:]:
        q, k, v, hs = _tc_comb_qkv(agg, hs, lp)
        agg = _sc_layer(q, k, v, ksrc, kdst, cnts)
    return _tc_out(agg, hs, params['out_W1'], params['out_b1'],
                   params['out_W2'], params['out_b2'],
                   params['out_W3'], params['out_b3'])
